# Initial kernel scaffold; baseline (speedup 1.0000x reference)
import jax
import jax.numpy as jnp
from jax.experimental import pallas as pl


def kernel(edge_index, edge_attr, init):
    raise NotImplementedError("write your pallas kernel here")


# speedup vs baseline: 1.5222x; 1.5222x over previous
import jax
import jax.numpy as jnp
from jax.experimental import pallas as pl
from jax.experimental.pallas import tpu as pltpu
from jax.experimental.pallas import tpu_sc as plsc

_W = 64


def kernel(edge_index, edge_attr, init):
    n_edges, d_edge = edge_attr.shape
    n_nodes, d_feat = init.shape
    d_out = d_edge + 2 * d_feat
    assert n_edges % _W == 0
    lane = d_edge

    idx = edge_index.astype(jnp.int32)
    src = idx[0].reshape(n_edges // _W, 1, _W)
    dst = idx[1].reshape(n_edges // _W, 1, _W)

    mesh = plsc.VectorSubcoreMesh(core_axis_name="c", subcore_axis_name="s")

    @pl.kernel(
        out_type=jax.ShapeDtypeStruct((n_edges, d_out), jnp.float32),
        mesh=mesh,
        scratch_types=[
            pltpu.VMEM((_W, d_feat), jnp.float32),
            pltpu.VMEM((_W, d_feat), jnp.float32),
        ],
    )
    def k(init_hbm, src_hbm, dst_hbm, attr_hbm, o_hbm, rs_v, rd_v):
        def body(src_v, dst_v, attr_v, o_v):
            pltpu.sync_copy(init_hbm.at[src_v.at[0, 0]], rs_v)
            pltpu.sync_copy(init_hbm.at[dst_v.at[0, 0]], rd_v)

            @pl.loop(0, _W)
            def _(i):
                o_v[i, pl.ds(0, lane)] = attr_v[i, pl.ds(0, lane)]
                for kk in range(d_feat // lane):
                    o_v[i, pl.ds(d_edge + kk * lane, lane)] = rs_v[
                        i, pl.ds(kk * lane, lane)
                    ]
                    o_v[i, pl.ds(d_edge + d_feat + kk * lane, lane)] = rd_v[
                        i, pl.ds(kk * lane, lane)
                    ]

        pltpu.emit_pipeline(
            body,
            grid=(n_edges // _W,),
            in_specs=[
                pl.BlockSpec((1, 1, _W), lambda i: (i, 0, 0)),
                pl.BlockSpec((1, 1, _W), lambda i: (i, 0, 0)),
                pl.BlockSpec((_W, d_edge), lambda i: (i, 0)),
            ],
            out_specs=[pl.BlockSpec((_W, d_out), lambda i: (i, 0))],
            core_axis_name=("c", "s"),
            dimension_semantics=(pltpu.PARALLEL,),
        )(src_hbm, dst_hbm, attr_hbm, o_hbm)

    return k(init, src, dst, edge_attr)


# speedup vs baseline: 2.1385x; 1.4048x over previous
import jax
import jax.numpy as jnp
from jax.experimental import pallas as pl
from jax.experimental.pallas import tpu as pltpu
from jax.experimental.pallas import tpu_sc as plsc

_W = 128
_B = 1280


def kernel(edge_index, edge_attr, init):
    n_edges, d_edge = edge_attr.shape
    n_nodes, d_feat = init.shape
    d_out = d_edge + 2 * d_feat
    assert n_edges % _W == 0 and n_edges % _B == 0

    idx = edge_index.astype(jnp.int32)
    src = idx[0].reshape(n_edges // _W, 1, _W)
    dst = idx[1].reshape(n_edges // _W, 1, _W)

    mesh = plsc.VectorSubcoreMesh(core_axis_name="c", subcore_axis_name="s")

    @pl.kernel(
        out_type=jax.ShapeDtypeStruct((n_edges, 2 * d_feat), jnp.float32),
        mesh=mesh,
    )
    def gather_k(init_hbm, src_hbm, dst_hbm, g_hbm):
        def body(src_v, dst_v, g_v):
            pltpu.sync_copy(
                init_hbm.at[src_v.at[0, 0]], g_v.at[:, pl.ds(0, d_feat)]
            )
            pltpu.sync_copy(
                init_hbm.at[dst_v.at[0, 0]], g_v.at[:, pl.ds(d_feat, d_feat)]
            )

        pltpu.emit_pipeline(
            body,
            grid=(n_edges // _W,),
            in_specs=[
                pl.BlockSpec((1, 1, _W), lambda i: (i, 0, 0)),
                pl.BlockSpec((1, 1, _W), lambda i: (i, 0, 0)),
            ],
            out_specs=[pl.BlockSpec((_W, 2 * d_feat), lambda i: (i, 0))],
            core_axis_name=("c", "s"),
            dimension_semantics=(pltpu.PARALLEL,),
        )(src_hbm, dst_hbm, g_hbm)

    g = gather_k(init, src, dst)

    def concat_body(attr_ref, g_ref, o_ref):
        o_ref[:, pl.ds(0, d_edge)] = attr_ref[...]
        o_ref[:, pl.ds(d_edge, 2 * d_feat)] = g_ref[...]

    return pl.pallas_call(
        concat_body,
        grid=(n_edges // _B,),
        in_specs=[
            pl.BlockSpec((_B, d_edge), lambda i: (i, 0)),
            pl.BlockSpec((_B, 2 * d_feat), lambda i: (i, 0)),
        ],
        out_specs=pl.BlockSpec((_B, d_out), lambda i: (i, 0)),
        out_shape=jax.ShapeDtypeStruct((n_edges, d_out), jnp.float32),
    )(edge_attr, g)


# speedup vs baseline: 2.1413x; 1.0013x over previous
import jax
import jax.numpy as jnp
from jax.experimental import pallas as pl
from jax.experimental.pallas import tpu as pltpu
from jax.experimental.pallas import tpu_sc as plsc

_W = 128
_B = 1280


def kernel(edge_index, edge_attr, init):
    n_edges, d_edge = edge_attr.shape
    n_nodes, d_feat = init.shape
    d_out = d_edge + 2 * d_feat
    assert n_edges % _W == 0 and n_edges % _B == 0

    idx = edge_index.astype(jnp.int32)
    src = idx[0].reshape(n_edges // _W, 1, _W)
    dst = idx[1].reshape(n_edges // _W, 1, _W)

    mesh = plsc.VectorSubcoreMesh(core_axis_name="c", subcore_axis_name="s")

    @pl.kernel(
        out_type=jax.ShapeDtypeStruct((n_edges, 2 * d_feat), jnp.float32),
        mesh=mesh,
    )
    def gather_k(init_hbm, src_hbm, dst_hbm, g_hbm):
        def body(src_v, dst_v, g_v):
            pltpu.sync_copy(
                init_hbm.at[src_v.at[0, 0]], g_v.at[:, pl.ds(0, d_feat)]
            )
            pltpu.sync_copy(
                init_hbm.at[dst_v.at[0, 0]], g_v.at[:, pl.ds(d_feat, d_feat)]
            )

        pltpu.emit_pipeline(
            body,
            grid=(n_edges // _W,),
            in_specs=[
                pl.BlockSpec((1, 1, _W), lambda i: (i, 0, 0)),
                pl.BlockSpec((1, 1, _W), lambda i: (i, 0, 0)),
            ],
            out_specs=[pl.BlockSpec((_W, 2 * d_feat), lambda i: (i, 0))],
            core_axis_name=("c", "s"),
            dimension_semantics=(pltpu.PARALLEL,),
        )(src_hbm, dst_hbm, g_hbm)

    g = gather_k(init, src, dst)

    def concat_body(attr_ref, g_ref, o_ref):
        o_ref[...] = jnp.concatenate([attr_ref[...], g_ref[...]], axis=-1)

    return pl.pallas_call(
        concat_body,
        grid=(n_edges // _B,),
        in_specs=[
            pl.BlockSpec((_B, d_edge), lambda i: (i, 0)),
            pl.BlockSpec((_B, 2 * d_feat), lambda i: (i, 0)),
        ],
        out_specs=pl.BlockSpec((_B, d_out), lambda i: (i, 0)),
        out_shape=jax.ShapeDtypeStruct((n_edges, d_out), jnp.float32),
    )(edge_attr, g)


# speedup vs baseline: 2.2150x; 1.0344x over previous
import jax
import jax.numpy as jnp
from jax.experimental import pallas as pl
from jax.experimental.pallas import tpu as pltpu
from jax.experimental.pallas import tpu_sc as plsc

_W = 128
_B = 1600
_K = 4


def _sc_gather(init, src, dst, n_edges, d_feat):
    mesh = plsc.VectorSubcoreMesh(core_axis_name="c", subcore_axis_name="s")

    @pl.kernel(
        out_type=jax.ShapeDtypeStruct((n_edges, 2 * d_feat), jnp.float32),
        mesh=mesh,
    )
    def gather_k(init_hbm, src_hbm, dst_hbm, g_hbm):
        def body(src_v, dst_v, g_v):
            pltpu.sync_copy(
                init_hbm.at[src_v.at[0, 0]], g_v.at[:, pl.ds(0, d_feat)]
            )
            pltpu.sync_copy(
                init_hbm.at[dst_v.at[0, 0]], g_v.at[:, pl.ds(d_feat, d_feat)]
            )

        pltpu.emit_pipeline(
            body,
            grid=(n_edges // _W,),
            in_specs=[
                pl.BlockSpec((1, 1, _W), lambda i: (i, 0, 0)),
                pl.BlockSpec((1, 1, _W), lambda i: (i, 0, 0)),
            ],
            out_specs=[pl.BlockSpec((_W, 2 * d_feat), lambda i: (i, 0))],
            core_axis_name=("c", "s"),
            dimension_semantics=(pltpu.PARALLEL,),
        )(src_hbm, dst_hbm, g_hbm)

    return gather_k(init, src, dst)


def kernel(edge_index, edge_attr, init):
    n_edges, d_edge = edge_attr.shape
    n_nodes, d_feat = init.shape
    d_out = d_edge + 2 * d_feat
    chunk = n_edges // _K
    assert chunk % _W == 0 and chunk % _B == 0

    idx = edge_index.astype(jnp.int32)
    src = idx[0].reshape(n_edges // _W, 1, _W)
    dst = idx[1].reshape(n_edges // _W, 1, _W)
    cw = chunk // _W

    gs = [
        _sc_gather(
            init,
            src[k * cw : (k + 1) * cw],
            dst[k * cw : (k + 1) * cw],
            chunk,
            d_feat,
        )
        for k in range(_K)
    ]

    def concat_body(attr_ref, g_ref, _, o_ref):
        o_ref[...] = jnp.concatenate([attr_ref[...], g_ref[...]], axis=-1)

    def concat_first(attr_ref, g_ref, o_ref):
        o_ref[...] = jnp.concatenate([attr_ref[...], g_ref[...]], axis=-1)

    cb = chunk // _B
    out = pl.pallas_call(
        concat_first,
        grid=(cb,),
        in_specs=[
            pl.BlockSpec((_B, d_edge), lambda i: (i, 0)),
            pl.BlockSpec((_B, 2 * d_feat), lambda i: (i, 0)),
        ],
        out_specs=pl.BlockSpec((_B, d_out), lambda i: (i, 0)),
        out_shape=jax.ShapeDtypeStruct((n_edges, d_out), jnp.float32),
    )(edge_attr[:chunk], gs[0])

    for k in range(1, _K):
        out = pl.pallas_call(
            concat_body,
            grid=(cb,),
            in_specs=[
                pl.BlockSpec((_B, d_edge), lambda i, k=k: (k * cb + i, 0)),
                pl.BlockSpec((_B, 2 * d_feat), lambda i: (i, 0)),
                pl.BlockSpec(memory_space=pltpu.MemorySpace.HBM),
            ],
            out_specs=pl.BlockSpec(
                (_B, d_out), lambda i, k=k: (k * cb + i, 0)
            ),
            out_shape=jax.ShapeDtypeStruct((n_edges, d_out), jnp.float32),
            input_output_aliases={2: 0},
        )(edge_attr, gs[k], out)

    return out


# speedup vs baseline: 2.2521x; 1.0168x over previous
import jax
import jax.numpy as jnp
from jax.experimental import pallas as pl
from jax.experimental.pallas import tpu as pltpu
from jax.experimental.pallas import tpu_sc as plsc

_W = 128
_B = 1600
_CHUNKS = (32000, 96000, 96000, 96000)


def _sc_gather(init, src, dst, n_edges, d_feat):
    mesh = plsc.VectorSubcoreMesh(core_axis_name="c", subcore_axis_name="s")

    @pl.kernel(
        out_type=jax.ShapeDtypeStruct((n_edges, 2 * d_feat), jnp.float32),
        mesh=mesh,
    )
    def gather_k(init_hbm, src_hbm, dst_hbm, g_hbm):
        def body(src_v, dst_v, g_v):
            pltpu.sync_copy(
                init_hbm.at[src_v.at[0, 0]], g_v.at[:, pl.ds(0, d_feat)]
            )
            pltpu.sync_copy(
                init_hbm.at[dst_v.at[0, 0]], g_v.at[:, pl.ds(d_feat, d_feat)]
            )

        pltpu.emit_pipeline(
            body,
            grid=(n_edges // _W,),
            in_specs=[
                pl.BlockSpec((1, 1, _W), lambda i: (i, 0, 0)),
                pl.BlockSpec((1, 1, _W), lambda i: (i, 0, 0)),
            ],
            out_specs=[pl.BlockSpec((_W, 2 * d_feat), lambda i: (i, 0))],
            core_axis_name=("c", "s"),
            dimension_semantics=(pltpu.PARALLEL,),
        )(src_hbm, dst_hbm, g_hbm)

    return gather_k(init, src, dst)


def kernel(edge_index, edge_attr, init):
    n_edges, d_edge = edge_attr.shape
    n_nodes, d_feat = init.shape
    d_out = d_edge + 2 * d_feat
    assert sum(_CHUNKS) == n_edges
    assert all(c % _W == 0 and c % _B == 0 for c in _CHUNKS)

    idx = edge_index.astype(jnp.int32)
    src = idx[0].reshape(n_edges // _W, 1, _W)
    dst = idx[1].reshape(n_edges // _W, 1, _W)

    starts = [sum(_CHUNKS[:k]) for k in range(len(_CHUNKS))]
    gs = [
        _sc_gather(
            init,
            src[s // _W : (s + c) // _W],
            dst[s // _W : (s + c) // _W],
            c,
            d_feat,
        )
        for s, c in zip(starts, _CHUNKS)
    ]

    def concat_body(attr_ref, g_ref, _, o_ref):
        o_ref[...] = jnp.concatenate([attr_ref[...], g_ref[...]], axis=-1)

    def concat_first(attr_ref, g_ref, o_ref):
        o_ref[...] = jnp.concatenate([attr_ref[...], g_ref[...]], axis=-1)

    out = pl.pallas_call(
        concat_first,
        grid=(_CHUNKS[0] // _B,),
        in_specs=[
            pl.BlockSpec((_B, d_edge), lambda i: (i, 0)),
            pl.BlockSpec((_B, 2 * d_feat), lambda i: (i, 0)),
        ],
        out_specs=pl.BlockSpec((_B, d_out), lambda i: (i, 0)),
        out_shape=jax.ShapeDtypeStruct((n_edges, d_out), jnp.float32),
    )(edge_attr[: _CHUNKS[0]], gs[0])

    for k in range(1, len(_CHUNKS)):
        b0 = starts[k] // _B
        out = pl.pallas_call(
            concat_body,
            grid=(_CHUNKS[k] // _B,),
            in_specs=[
                pl.BlockSpec((_B, d_edge), lambda i, b0=b0: (b0 + i, 0)),
                pl.BlockSpec((_B, 2 * d_feat), lambda i: (i, 0)),
                pl.BlockSpec(memory_space=pltpu.MemorySpace.HBM),
            ],
            out_specs=pl.BlockSpec(
                (_B, d_out), lambda i, b0=b0: (b0 + i, 0)
            ),
            out_shape=jax.ShapeDtypeStruct((n_edges, d_out), jnp.float32),
            input_output_aliases={2: 0},
        )(edge_attr, gs[k], out)

    return out


# speedup vs baseline: 2.7698x; 1.2299x over previous
import jax
import jax.numpy as jnp
from jax import lax
from jax.experimental import pallas as pl
from jax.experimental.pallas import tpu as pltpu
from jax.experimental.pallas import tpu_sc as plsc

_W = 64
_NW = 32


def kernel(edge_index, edge_attr, init):
    n_edges, d_edge = edge_attr.shape
    n_nodes, d_feat = init.shape
    d_out = d_edge + 2 * d_feat
    nb = n_edges // _W
    assert n_edges % _W == 0
    assert nb // _NW >= 2
    lane = d_edge
    spf = d_feat // lane
    trips = (nb + _NW - 1) // _NW
    trips += trips % 2

    idx = edge_index.astype(jnp.int32)
    src = idx[0].reshape(nb, 1, _W)
    dst = idx[1].reshape(nb, 1, _W)

    mesh = plsc.VectorSubcoreMesh(core_axis_name="c", subcore_axis_name="s")

    f32 = jnp.float32
    @pl.kernel(
        out_type=jax.ShapeDtypeStruct((n_edges, d_out), f32),
        mesh=mesh,
        scratch_types=[
            pltpu.VMEM((1, _W), jnp.int32),
            pltpu.VMEM((1, _W), jnp.int32),
            pltpu.VMEM((1, _W), jnp.int32),
            pltpu.VMEM((1, _W), jnp.int32),
            pltpu.VMEM((_W, d_feat), f32),
            pltpu.VMEM((_W, d_feat), f32),
            pltpu.VMEM((_W, d_feat), f32),
            pltpu.VMEM((_W, d_feat), f32),
            pltpu.VMEM((_W, d_edge), f32),
            pltpu.VMEM((_W, d_edge), f32),
            pltpu.VMEM((_W, d_out), f32),
            pltpu.VMEM((_W, d_out), f32),
        ]
        + [pltpu.SemaphoreType.DMA] * 12,
    )
    def k(init_hbm, src_hbm, dst_hbm, attr_hbm, o_hbm, *sc):
        is_ = sc[0:2]
        id_ = sc[2:4]
        rs_ = sc[4:6]
        rd_ = sc[6:8]
        at_ = sc[8:10]
        o_ = sc[10:12]
        s_is = sc[12:14]
        s_id = sc[14:16]
        s_gs = sc[16:18]
        s_gd = sc[18:20]
        s_at = sc[20:22]
        s_out = sc[22:24]

        wid = lax.axis_index("s") * 2 + lax.axis_index("c")

        def issue_idx(b, p):
            pltpu.async_copy(src_hbm.at[b], is_[p], s_is[p])
            pltpu.async_copy(dst_hbm.at[b], id_[p], s_id[p])

        def wait_idx(p):
            pltpu.make_async_copy(src_hbm.at[0], is_[p], s_is[p]).wait()
            pltpu.make_async_copy(dst_hbm.at[0], id_[p], s_id[p]).wait()

        def issue_gather(b, p):
            pltpu.async_copy(init_hbm.at[is_[p].at[0]], rs_[p], s_gs[p])
            pltpu.async_copy(init_hbm.at[id_[p].at[0]], rd_[p], s_gd[p])
            pltpu.async_copy(attr_hbm.at[pl.ds(b * _W, _W)], at_[p], s_at[p])

        def wait_gather(p):
            pltpu.make_async_copy(
                init_hbm.at[is_[p].at[0]], rs_[p], s_gs[p]
            ).wait()
            pltpu.make_async_copy(
                init_hbm.at[id_[p].at[0]], rd_[p], s_gd[p]
            ).wait()
            pltpu.make_async_copy(
                attr_hbm.at[pl.ds(0, _W)], at_[p], s_at[p]
            ).wait()

        def issue_out(b, p):
            pltpu.async_copy(o_[p], o_hbm.at[pl.ds(b * _W, _W)], s_out[p])

        def wait_out(p):
            pltpu.make_async_copy(
                o_[p], o_hbm.at[pl.ds(0, _W)], s_out[p]
            ).wait()

        def assemble(p):
            at_v, rs_v, rd_v, o_v = at_[p], rs_[p], rd_[p], o_[p]

            @pl.loop(0, _W)
            def _(i):
                vals = [at_v[i, pl.ds(0, lane)]]
                vals += [rs_v[i, pl.ds(kk * lane, lane)] for kk in range(spf)]
                vals += [rd_v[i, pl.ds(kk * lane, lane)] for kk in range(spf)]
                for j, v in enumerate(vals):
                    o_v[i, pl.ds(j * lane, lane)] = v

        b0 = wid
        b1 = wid + _NW

        @pl.when(b0 < nb)
        def _():
            issue_idx(b0, 0)

        @pl.when(b1 < nb)
        def _():
            issue_idx(b1, 1)

        @pl.when(b0 < nb)
        def _():
            wait_idx(0)
            issue_gather(b0, 0)

        def trip(t, p):
            b = wid + t * _NW
            bn = b + _NW
            bnn = b + 2 * _NW
            q = 1 - p

            @pl.when(b < nb)
            def _():
                @pl.when(bn < nb)
                def _():
                    wait_idx(q)
                    issue_gather(bn, q)

                wait_gather(p)

                @pl.when(bnn < nb)
                def _():
                    issue_idx(bnn, p)

                @pl.when(t >= 2)
                def _():
                    wait_out(p)

                assemble(p)
                issue_out(b, p)

        @pl.loop(0, trips, step=2)
        def _(t):
            trip(t, 0)
            trip(t + 1, 1)

        wait_out(0)
        wait_out(1)

    return k(init, src, dst, edge_attr)
